# baseline (speedup 1.0000x reference)
import functools

import jax
import jax.numpy as jnp
from jax import lax
from jax.experimental import pallas as pl
from jax.experimental.pallas import tpu as pltpu
from jax.experimental.pallas import tpu_sc as plsc

T, D = 2048, 768
E, K = 8, 2
FF = 4 * D

BT = 128
G = T * K // BT + E
R = G * BT

NC, NS = 2, 16
NW = NC * NS


def _gate_route_body(h_ref, p0_ref, p1_ref, w1_ref, w2_ref, meta_ref):
    h = h_ref[...]
    iota = lax.broadcasted_iota(jnp.int32, (T, E), 1)
    m1 = jnp.max(h, axis=1)
    a1 = jnp.min(jnp.where(h == m1[:, None], iota, E), axis=1)
    hm = jnp.where(iota == a1[:, None], -jnp.inf, h)
    m2 = jnp.max(hm, axis=1)
    a2 = jnp.min(jnp.where(hm == m2[:, None], iota, E), axis=1)
    d = jnp.exp(m2 - m1)
    w1_ref[...] = 1.0 / (1.0 + d)
    w2_ref[...] = d / (1.0 + d)
    oh0 = (iota == a1[:, None]).astype(jnp.float32)
    oh1 = (iota == a2[:, None]).astype(jnp.float32)
    cb = 256
    tri = (lax.broadcasted_iota(jnp.int32, (cb, cb), 0)
           >= lax.broadcasted_iota(jnp.int32, (cb, cb), 1)).astype(jnp.bfloat16)

    def csum_incl(ohf):
        outs, base = [], jnp.zeros((E,), jnp.float32)
        for blk in range(T // cb):
            ob = ohf[blk * cb:(blk + 1) * cb]
            loc = lax.dot_general(
                tri, ob.astype(jnp.bfloat16), (((1,), (0,)), ((), ())),
                preferred_element_type=jnp.float32)
            outs.append(loc + base[None, :])
            base = base + jnp.sum(ob, axis=0)
        return jnp.concatenate(outs, axis=0)

    cs0 = csum_incl(oh0) - oh0
    cs1 = csum_incl(oh1) - oh1
    tot = cs0[-1] + oh0[-1] + cs1[-1] + oh1[-1]
    alig = jnp.floor((tot + (BT - 1)) / BT) * BT
    tri8 = (lax.broadcasted_iota(jnp.int32, (E, E), 0)
            >= lax.broadcasted_iota(jnp.int32, (E, E), 1)).astype(jnp.float32)
    cum = jnp.sum(tri8 * alig[None, :], axis=1)
    offs = cum - alig
    rank0 = jnp.sum((cs0 + cs1) * oh0, axis=1)
    rank1 = jnp.sum((cs0 + oh0 + cs1) * oh1, axis=1)
    off0 = jnp.sum(oh0 * offs[None, :], axis=1)
    off1 = jnp.sum(oh1 * offs[None, :], axis=1)
    p0_ref[...] = (rank0 + off0).astype(jnp.int32)
    p1_ref[...] = (rank1 + off1).astype(jnp.int32)
    bidx = (lax.broadcasted_iota(jnp.int32, (G, E), 0) * BT).astype(jnp.float32)
    be = jnp.sum((cum[None, :] <= bidx).astype(jnp.int32), axis=1)
    be = jnp.minimum(be, E - 1)
    used = (cum[E - 1] / BT).astype(jnp.int32)
    meta_ref[...] = jnp.concatenate([be, used[None]])


def _gate_route(h):
    return pl.pallas_call(
        _gate_route_body,
        out_shape=[
            jax.ShapeDtypeStruct((T,), jnp.int32),
            jax.ShapeDtypeStruct((T,), jnp.int32),
            jax.ShapeDtypeStruct((T,), jnp.float32),
            jax.ShapeDtypeStruct((T,), jnp.float32),
            jax.ShapeDtypeStruct((G + 1,), jnp.int32),
        ],
    )(h)


def _ffn_body(m_ref, xb_ref, p0_ref, p1_ref,
              W1_ref, b1_ref, W2_ref, b2_ref, out_ref):
    b = pl.program_id(0)

    @pl.when(b < m_ref[G])
    def _():
        slot = lax.broadcasted_iota(jnp.int32, (BT, T), 0) + b * BT
        eq0 = slot == p0_ref[...][None, :]
        eq1 = slot == p1_ref[...][None, :]
        sel = (eq0 | eq1).astype(jnp.bfloat16)
        xs = lax.dot_general(
            sel, xb_ref[...], (((1,), (0,)), ((), ())),
            preferred_element_type=jnp.float32).astype(jnp.bfloat16)
        h1 = lax.dot_general(
            xs, W1_ref[0].astype(jnp.bfloat16), (((1,), (0,)), ((), ())),
            preferred_element_type=jnp.float32)
        h1 = jnp.maximum(h1 + b1_ref[0, 0][None, :], 0.0)
        y = lax.dot_general(
            h1.astype(jnp.bfloat16), W2_ref[0].astype(jnp.bfloat16),
            (((1,), (0,)), ((), ())),
            preferred_element_type=jnp.float32)
        out_ref[...] = y + b2_ref[0, 0][None, :]


def _ffn(meta, xb, p0, p1, W1, b1, W2, b2):
    grid_spec = pltpu.PrefetchScalarGridSpec(
        num_scalar_prefetch=1,
        grid=(G,),
        in_specs=[
            pl.BlockSpec((T, D), lambda b, m: (0, 0)),
            pl.BlockSpec((T,), lambda b, m: (0,)),
            pl.BlockSpec((T,), lambda b, m: (0,)),
            pl.BlockSpec((1, D, FF), lambda b, m: (m[b], 0, 0)),
            pl.BlockSpec((1, 1, FF), lambda b, m: (m[b], 0, 0)),
            pl.BlockSpec((1, FF, D), lambda b, m: (m[b], 0, 0)),
            pl.BlockSpec((1, 1, D), lambda b, m: (m[b], 0, 0)),
        ],
        out_specs=pl.BlockSpec((BT, D), lambda b, m: (b, 0)),
    )
    return pl.pallas_call(
        _ffn_body,
        grid_spec=grid_spec,
        out_shape=jax.ShapeDtypeStruct((R, D), jnp.float32),
        compiler_params=pltpu.CompilerParams(
            dimension_semantics=("arbitrary",)),
    )(meta, xb, p0, p1,
      W1, b1.reshape(E, 1, FF), W2, b2.reshape(E, 1, D))


_TW = T // NW
_CC = 32


def _sc_combine(hw, p0, p1, w0, w1):
    mesh = plsc.VectorSubcoreMesh(core_axis_name="c", subcore_axis_name="s")
    nch = _TW // _CC

    @functools.partial(
        pl.kernel, mesh=mesh,
        out_type=jax.ShapeDtypeStruct((T, D), jnp.float32),
        scratch_types=[
            pltpu.VMEM((nch, _CC), jnp.int32),
            pltpu.VMEM((nch, _CC), jnp.int32),
            pltpu.VMEM((_TW + 16,), jnp.float32),
            pltpu.VMEM((_TW + 16,), jnp.float32),
            pltpu.VMEM((_CC, D), jnp.float32),
            pltpu.VMEM((_CC, D), jnp.float32),
            pltpu.VMEM((_CC, D), jnp.float32),
            pltpu.VMEM((_CC, D), jnp.float32),
            pltpu.VMEM((_CC, D), jnp.float32),
            pltpu.SemaphoreType.DMA,
            pltpu.SemaphoreType.DMA,
            pltpu.SemaphoreType.DMA,
            pltpu.SemaphoreType.DMA,
        ],
    )
    def k(hw_hbm, p0_hbm, p1_hbm, w0_hbm, w1_hbm, out_hbm,
          i0_v, i1_v, w0_v, w1_v, r00, r10, r01, r11, out_v,
          s00, s10, s01, s11):
        wid = lax.axis_index("s") * NC + lax.axis_index("c")
        tbase = wid * _TW
        pltpu.sync_copy(p0_hbm.at[wid], i0_v)
        pltpu.sync_copy(p1_hbm.at[wid], i1_v)
        pltpu.sync_copy(w0_hbm.at[pl.ds(tbase, _TW)], w0_v.at[pl.ds(0, _TW)])
        pltpu.sync_copy(w1_hbm.at[pl.ds(tbase, _TW)], w1_v.at[pl.ds(0, _TW)])
        r0b, r1b = (r00, r01), (r10, r11)
        h0 = [pltpu.async_copy(hw_hbm.at[i0_v.at[c]], r0b[c], s)
              for c, s in ((0, s00), (1, s01))]
        h1 = [pltpu.async_copy(hw_hbm.at[i1_v.at[c]], r1b[c], s)
              for c, s in ((0, s10), (1, s11))]
        for c in range(nch):
            h0[c].wait()
            h1[c].wait()
            r0v, r1v = r0b[c], r1b[c]
            ct = c * _CC

            def body(t, carry):
                wv0 = w0_v[pl.ds(ct + t, 16)][0]
                wv1 = w1_v[pl.ds(ct + t, 16)][0]
                for dch in range(D // 16):
                    sl = pl.ds(dch * 16, 16)
                    out_v[t, sl] = r0v[t, sl] * wv0 + r1v[t, sl] * wv1
                return carry

            lax.fori_loop(0, _CC, body, 0)
            pltpu.sync_copy(out_v, out_hbm.at[pl.ds(tbase + ct, _CC)])

    return k(hw, p0, p1, w0, w1)


def kernel(x, Wg, bg, Wn, bn, W1, b1, W2, b2):
    x2 = x[0]
    noise = jax.random.normal(jax.random.PRNGKey(42), (1, T, E),
                              dtype=jnp.float32)
    h_logits = (x @ Wg + bg + noise * jax.nn.softplus(x @ Wn + bn))[0]
    p0, p1, gw0, gw1, meta = _gate_route(h_logits)
    hw = _ffn(meta, x2.astype(jnp.bfloat16), p0, p1, W1, b1, W2, b2)
    out2 = _sc_combine(hw, p0.reshape(NW, _TW // _CC, _CC),
                       p1.reshape(NW, _TW // _CC, _CC), gw0, gw1)
    return out2[None, :, :]


# speedup vs baseline: 1.0734x; 1.0734x over previous
import functools

import jax
import jax.numpy as jnp
from jax import lax
from jax.experimental import pallas as pl
from jax.experimental.pallas import tpu as pltpu
from jax.experimental.pallas import tpu_sc as plsc

T, D = 2048, 768
E, K = 8, 2
FF = 4 * D

BT = 256
G = T * K // BT + E
R = G * BT

NC, NS = 2, 16
NW = NC * NS


def _gate_route_body(h_ref, p0_ref, p1_ref, w1_ref, w2_ref, meta_ref):
    h = h_ref[...]
    iota = lax.broadcasted_iota(jnp.int32, (T, E), 1)
    m1 = jnp.max(h, axis=1)
    a1 = jnp.min(jnp.where(h == m1[:, None], iota, E), axis=1)
    hm = jnp.where(iota == a1[:, None], -jnp.inf, h)
    m2 = jnp.max(hm, axis=1)
    a2 = jnp.min(jnp.where(hm == m2[:, None], iota, E), axis=1)
    d = jnp.exp(m2 - m1)
    w1_ref[...] = 1.0 / (1.0 + d)
    w2_ref[...] = d / (1.0 + d)
    oh0 = (iota == a1[:, None]).astype(jnp.float32)
    oh1 = (iota == a2[:, None]).astype(jnp.float32)
    cb = 256
    tri = (lax.broadcasted_iota(jnp.int32, (cb, cb), 0)
           >= lax.broadcasted_iota(jnp.int32, (cb, cb), 1)).astype(jnp.bfloat16)

    def csum_incl(ohf):
        outs, base = [], jnp.zeros((E,), jnp.float32)
        for blk in range(T // cb):
            ob = ohf[blk * cb:(blk + 1) * cb]
            loc = lax.dot_general(
                tri, ob.astype(jnp.bfloat16), (((1,), (0,)), ((), ())),
                preferred_element_type=jnp.float32)
            outs.append(loc + base[None, :])
            base = base + jnp.sum(ob, axis=0)
        return jnp.concatenate(outs, axis=0)

    cs0 = csum_incl(oh0) - oh0
    cs1 = csum_incl(oh1) - oh1
    tot = cs0[-1] + oh0[-1] + cs1[-1] + oh1[-1]
    alig = jnp.floor((tot + (BT - 1)) / BT) * BT
    tri8 = (lax.broadcasted_iota(jnp.int32, (E, E), 0)
            >= lax.broadcasted_iota(jnp.int32, (E, E), 1)).astype(jnp.float32)
    cum = jnp.sum(tri8 * alig[None, :], axis=1)
    offs = cum - alig
    rank0 = jnp.sum((cs0 + cs1) * oh0, axis=1)
    rank1 = jnp.sum((cs0 + oh0 + cs1) * oh1, axis=1)
    off0 = jnp.sum(oh0 * offs[None, :], axis=1)
    off1 = jnp.sum(oh1 * offs[None, :], axis=1)
    p0_ref[...] = (rank0 + off0).astype(jnp.int32)
    p1_ref[...] = (rank1 + off1).astype(jnp.int32)
    bidx = (lax.broadcasted_iota(jnp.int32, (G, E), 0) * BT).astype(jnp.float32)
    be = jnp.sum((cum[None, :] <= bidx).astype(jnp.int32), axis=1)
    be = jnp.minimum(be, E - 1)
    used = (cum[E - 1] / BT).astype(jnp.int32)
    meta_ref[...] = jnp.concatenate([be, used[None]])


def _gate_route(h):
    return pl.pallas_call(
        _gate_route_body,
        out_shape=[
            jax.ShapeDtypeStruct((T,), jnp.int32),
            jax.ShapeDtypeStruct((T,), jnp.int32),
            jax.ShapeDtypeStruct((T,), jnp.float32),
            jax.ShapeDtypeStruct((T,), jnp.float32),
            jax.ShapeDtypeStruct((G + 1,), jnp.int32),
        ],
    )(h)


def _ffn_body(m_ref, xb_ref, p0_ref, p1_ref,
              W1_ref, b1_ref, W2_ref, b2_ref, out_ref):
    b = pl.program_id(0)

    @pl.when(b < m_ref[G])
    def _():
        slot = lax.broadcasted_iota(jnp.int32, (BT, T), 0) + b * BT
        eq0 = slot == p0_ref[...][None, :]
        eq1 = slot == p1_ref[...][None, :]
        sel = (eq0 | eq1).astype(jnp.bfloat16)
        xs = lax.dot_general(
            sel, xb_ref[...], (((1,), (0,)), ((), ())),
            preferred_element_type=jnp.float32).astype(jnp.bfloat16)
        h1 = lax.dot_general(
            xs, W1_ref[0].astype(jnp.bfloat16), (((1,), (0,)), ((), ())),
            preferred_element_type=jnp.float32)
        h1 = jnp.maximum(h1 + b1_ref[0, 0][None, :], 0.0)
        y = lax.dot_general(
            h1.astype(jnp.bfloat16), W2_ref[0].astype(jnp.bfloat16),
            (((1,), (0,)), ((), ())),
            preferred_element_type=jnp.float32)
        out_ref[...] = y + b2_ref[0, 0][None, :]


def _ffn(meta, xb, p0, p1, W1, b1, W2, b2):
    grid_spec = pltpu.PrefetchScalarGridSpec(
        num_scalar_prefetch=1,
        grid=(G,),
        in_specs=[
            pl.BlockSpec((T, D), lambda b, m: (0, 0)),
            pl.BlockSpec((T,), lambda b, m: (0,)),
            pl.BlockSpec((T,), lambda b, m: (0,)),
            pl.BlockSpec((1, D, FF), lambda b, m: (m[b], 0, 0)),
            pl.BlockSpec((1, 1, FF), lambda b, m: (m[b], 0, 0)),
            pl.BlockSpec((1, FF, D), lambda b, m: (m[b], 0, 0)),
            pl.BlockSpec((1, 1, D), lambda b, m: (m[b], 0, 0)),
        ],
        out_specs=pl.BlockSpec((BT, D), lambda b, m: (b, 0)),
    )
    return pl.pallas_call(
        _ffn_body,
        grid_spec=grid_spec,
        out_shape=jax.ShapeDtypeStruct((R, D), jnp.float32),
        compiler_params=pltpu.CompilerParams(
            dimension_semantics=("arbitrary",)),
    )(meta, xb, p0, p1,
      W1, b1.reshape(E, 1, FF), W2, b2.reshape(E, 1, D))


_TW = T // NW
_CC = 32


def _sc_combine(hw, p0, p1, w0, w1):
    mesh = plsc.VectorSubcoreMesh(core_axis_name="c", subcore_axis_name="s")
    nch = _TW // _CC

    @functools.partial(
        pl.kernel, mesh=mesh,
        out_type=jax.ShapeDtypeStruct((T, D), jnp.float32),
        scratch_types=[
            pltpu.VMEM((nch, _CC), jnp.int32),
            pltpu.VMEM((nch, _CC), jnp.int32),
            pltpu.VMEM((_TW + 16,), jnp.float32),
            pltpu.VMEM((_TW + 16,), jnp.float32),
            pltpu.VMEM((_CC, D), jnp.float32),
            pltpu.VMEM((_CC, D), jnp.float32),
            pltpu.VMEM((_CC, D), jnp.float32),
            pltpu.VMEM((_CC, D), jnp.float32),
            pltpu.VMEM((_CC, D), jnp.float32),
            pltpu.SemaphoreType.DMA,
            pltpu.SemaphoreType.DMA,
            pltpu.SemaphoreType.DMA,
            pltpu.SemaphoreType.DMA,
        ],
    )
    def k(hw_hbm, p0_hbm, p1_hbm, w0_hbm, w1_hbm, out_hbm,
          i0_v, i1_v, w0_v, w1_v, r00, r10, r01, r11, out_v,
          s00, s10, s01, s11):
        wid = lax.axis_index("s") * NC + lax.axis_index("c")
        tbase = wid * _TW
        pltpu.sync_copy(p0_hbm.at[wid], i0_v)
        pltpu.sync_copy(p1_hbm.at[wid], i1_v)
        pltpu.sync_copy(w0_hbm.at[pl.ds(tbase, _TW)], w0_v.at[pl.ds(0, _TW)])
        pltpu.sync_copy(w1_hbm.at[pl.ds(tbase, _TW)], w1_v.at[pl.ds(0, _TW)])
        r0b, r1b = (r00, r01), (r10, r11)
        h0 = [pltpu.async_copy(hw_hbm.at[i0_v.at[c]], r0b[c], s)
              for c, s in ((0, s00), (1, s01))]
        h1 = [pltpu.async_copy(hw_hbm.at[i1_v.at[c]], r1b[c], s)
              for c, s in ((0, s10), (1, s11))]
        for c in range(nch):
            h0[c].wait()
            h1[c].wait()
            r0v, r1v = r0b[c], r1b[c]
            ct = c * _CC

            def body(t, carry):
                wv0 = w0_v[pl.ds(ct + t, 16)][0]
                wv1 = w1_v[pl.ds(ct + t, 16)][0]
                for dch in range(D // 16):
                    sl = pl.ds(dch * 16, 16)
                    out_v[t, sl] = r0v[t, sl] * wv0 + r1v[t, sl] * wv1
                return carry

            lax.fori_loop(0, _CC, body, 0)
            pltpu.sync_copy(out_v, out_hbm.at[pl.ds(tbase + ct, _CC)])

    return k(hw, p0, p1, w0, w1)


def kernel(x, Wg, bg, Wn, bn, W1, b1, W2, b2):
    x2 = x[0]
    noise = jax.random.normal(jax.random.PRNGKey(42), (1, T, E),
                              dtype=jnp.float32)
    h_logits = (x @ Wg + bg + noise * jax.nn.softplus(x @ Wn + bn))[0]
    p0, p1, gw0, gw1, meta = _gate_route(h_logits)
    hw = _ffn(meta, x2.astype(jnp.bfloat16), p0, p1, W1, b1, W2, b2)
    out2 = _sc_combine(hw, p0.reshape(NW, _TW // _CC, _CC),
                       p1.reshape(NW, _TW // _CC, _CC), gw0, gw1)
    return out2[None, :, :]
